# baseline (speedup 1.0000x reference)
import functools

import jax
import jax.numpy as jnp
from jax import lax
from jax.experimental import pallas as pl
from jax.experimental.pallas import tpu as pltpu

_MAXLAG = 30
_NLAGS = 5
_PAD = 32


def _acf_kernel(x_ref, lags_ref, acc_ref, *, nb, t):
    i = pl.program_id(0)

    @pl.when(i == 0)
    def _init():
        for l in range(_MAXLAG):
            acc_ref[l] = 0.0

    x = x_ref[...]
    mu = jnp.mean(x, axis=1, keepdims=True)
    xc = x - mu
    var = jnp.sum(xc * xc, axis=1, keepdims=True)
    y = xc / (var + 1e-8)
    for lag in range(1, _MAXLAG + 1):
        contrib = jnp.sum(y[:, lag:] * xc[:, : t - lag])
        acc_ref[lag - 1] += contrib

    @pl.when(i == nb - 1)
    def _topk():
        def pick(k, _):
            def scan(l, carry):
                bv, bi = carry
                v = acc_ref[l]
                better = v > bv
                return (jnp.where(better, v, bv), jnp.where(better, l, bi))

            bv, bi = lax.fori_loop(0, _MAXLAG, scan, (jnp.float32(-jnp.inf), jnp.int32(0)))
            lags_ref[k] = bi + 1
            acc_ref[bi] = -jnp.inf
            return 0

        lax.fori_loop(0, _NLAGS, pick, 0)


def _feat_kernel(lags_ref, x_ref, out_ref, *, bb, t):
    x = x_ref[...]
    ti = lax.broadcasted_iota(jnp.int32, (bb, t), 1)
    feats = [x]
    for k in range(_NLAGS):
        lag = lags_ref[k]
        rolled = pltpu.roll(x, lag, 1)
        feats.append(jnp.where(ti < lag, 0.0, rolled))
    stacked = jnp.stack(feats, axis=-1)
    out_ref[...] = stacked.reshape(bb, t * (_NLAGS + 1))


def kernel(inputs):
    x = inputs
    b, t = x.shape
    bb1 = 256
    nb1 = b // bb1

    lags = pl.pallas_call(
        functools.partial(_acf_kernel, nb=nb1, t=t),
        grid=(nb1,),
        in_specs=[pl.BlockSpec((bb1, t), lambda i: (i, 0))],
        out_specs=pl.BlockSpec(memory_space=pltpu.SMEM),
        out_shape=jax.ShapeDtypeStruct((8,), jnp.int32),
        scratch_shapes=[pltpu.SMEM((_MAXLAG,), jnp.float32)],
    )(x)

    bb2 = 8
    nb2 = b // bb2
    out_flat = pl.pallas_call(
        functools.partial(_feat_kernel, bb=bb2, t=t),
        grid_spec=pltpu.PrefetchScalarGridSpec(
            num_scalar_prefetch=1,
            grid=(nb2,),
            in_specs=[pl.BlockSpec((bb2, t), lambda i, lags: (i, 0))],
            out_specs=pl.BlockSpec((bb2, t * (_NLAGS + 1)), lambda i, lags: (i, 0)),
        ),
        out_shape=jax.ShapeDtypeStruct((b, t * (_NLAGS + 1)), jnp.float32),
    )(lags, x)

    return out_flat.reshape(b, t, _NLAGS + 1)


# speedup vs baseline: 14.6848x; 14.6848x over previous
import functools

import jax
import jax.numpy as jnp
from jax import lax
from jax.experimental import pallas as pl
from jax.experimental.pallas import tpu as pltpu

_MAXLAG = 30
_NLAGS = 5


def _acf_part_kernel(x_ref, out_ref, *, t):
    x = x_ref[...]
    mu = jnp.mean(x, axis=1, keepdims=True)
    xc = x - mu
    var = jnp.sum(xc * xc, axis=1, keepdims=True)
    y = xc / (var + 1e-8)
    lane = lax.broadcasted_iota(jnp.int32, (1, 128), 1)
    acc = jnp.zeros((1, 128), jnp.float32)
    for lag in range(1, _MAXLAG + 1):
        contrib = jnp.sum(y[:, lag:] * xc[:, : t - lag])
        acc = acc + jnp.where(lane == lag - 1, contrib, 0.0)
    out_ref[...] = acc.reshape(1, 1, 128)


def _topk_kernel(parts_ref, lags_ref):
    v = jnp.sum(parts_ref[...], axis=0)
    lane = lax.broadcasted_iota(jnp.int32, (1, 128), 1)
    v = jnp.where(lane < _MAXLAG, v, -jnp.inf)
    for k in range(_NLAGS):
        m = jnp.max(v)
        idx = jnp.min(jnp.where(v == m, lane, 2 * _MAXLAG))
        lags_ref[k] = idx + 1
        v = jnp.where(lane == idx, -jnp.inf, v)


def _feat_kernel(lags_ref, x_ref, out_ref, *, bb, t):
    x = x_ref[...]
    out_ref[0] = x
    ti = lax.broadcasted_iota(jnp.int32, (bb, t), 1)
    for k in range(_NLAGS):
        lag = lags_ref[k]
        rolled = pltpu.roll(x, lag, 1)
        out_ref[k + 1] = jnp.where(ti < lag, 0.0, rolled)


def kernel(inputs):
    x = inputs
    b, t = x.shape
    bb1 = 256
    nb1 = b // bb1

    parts = pl.pallas_call(
        functools.partial(_acf_part_kernel, t=t),
        grid=(nb1,),
        in_specs=[pl.BlockSpec((bb1, t), lambda i: (i, 0))],
        out_specs=pl.BlockSpec((1, 1, 128), lambda i: (i, 0, 0)),
        out_shape=jax.ShapeDtypeStruct((nb1, 1, 128), jnp.float32),
        compiler_params=pltpu.CompilerParams(dimension_semantics=("parallel",)),
    )(x)

    lags = pl.pallas_call(
        _topk_kernel,
        in_specs=[pl.BlockSpec((nb1, 1, 128), lambda: (0, 0, 0))],
        out_specs=pl.BlockSpec(memory_space=pltpu.SMEM),
        out_shape=jax.ShapeDtypeStruct((8,), jnp.int32),
    )(parts)

    bb2 = 256
    nb2 = b // bb2
    planes = pl.pallas_call(
        functools.partial(_feat_kernel, bb=bb2, t=t),
        grid_spec=pltpu.PrefetchScalarGridSpec(
            num_scalar_prefetch=1,
            grid=(nb2,),
            in_specs=[pl.BlockSpec((bb2, t), lambda i, lags: (i, 0))],
            out_specs=pl.BlockSpec((_NLAGS + 1, bb2, t), lambda i, lags: (0, i, 0)),
        ),
        out_shape=jax.ShapeDtypeStruct((_NLAGS + 1, b, t), jnp.float32),
        compiler_params=pltpu.CompilerParams(dimension_semantics=("parallel",)),
    )(lags, x)

    return jnp.transpose(planes, (1, 2, 0))


# speedup vs baseline: 26.9381x; 1.8344x over previous
import functools

import jax
import jax.numpy as jnp
from jax import lax
from jax.experimental import pallas as pl
from jax.experimental.pallas import tpu as pltpu

_MAXLAG = 30
_NLAGS = 5
_PADT = 32


def _acf_part_kernel(x_ref, out_ref, xct_ref, yt_ref, *, t, bb):
    xt = jnp.transpose(x_ref[...])
    mu = jnp.mean(xt, axis=0, keepdims=True)
    xct = xt - mu
    var = jnp.sum(xct * xct, axis=0, keepdims=True)
    yt = xct / (var + 1e-8)
    xct_ref[:t, :] = xct
    yt_ref[:t, :] = yt
    yt_ref[t:, :] = jnp.zeros((_PADT, bb), jnp.float32)
    lane = lax.broadcasted_iota(jnp.int32, (1, 128), 1)
    acc = jnp.zeros((1, 128), jnp.float32)
    for lag in range(1, _MAXLAG + 1):
        prod = yt_ref[pl.ds(lag, t), :] * xct_ref[:t, :]
        r = jnp.sum(prod.reshape(t // 8, 8, bb), axis=0)
        contrib = jnp.sum(r)
        acc = acc + jnp.where(lane == lag - 1, contrib, 0.0)
    out_ref[...] = acc.reshape(1, 1, 128)


def _topk_kernel(parts_ref, lags_ref):
    v = jnp.sum(parts_ref[...], axis=0)
    lane = lax.broadcasted_iota(jnp.int32, (1, 128), 1)
    v = jnp.where(lane < _MAXLAG, v, -jnp.inf)
    for k in range(_NLAGS):
        m = jnp.max(v)
        idx = jnp.min(jnp.where(v == m, lane, 2 * _MAXLAG))
        lags_ref[k] = idx + 1
        v = jnp.where(lane == idx, -jnp.inf, v)


def _feat_kernel(lags_ref, x_ref, out_ref, *, bb, t):
    x = x_ref[...]
    out_ref[0] = x
    ti = lax.broadcasted_iota(jnp.int32, (bb, t), 1)
    for k in range(_NLAGS):
        lag = lags_ref[k]
        rolled = pltpu.roll(x, lag, 1)
        out_ref[k + 1] = jnp.where(ti < lag, 0.0, rolled)


def kernel(inputs):
    x = inputs
    b, t = x.shape
    bb1 = 256
    nb1 = b // bb1

    parts = pl.pallas_call(
        functools.partial(_acf_part_kernel, t=t, bb=bb1),
        grid=(nb1,),
        in_specs=[pl.BlockSpec((bb1, t), lambda i: (i, 0))],
        out_specs=pl.BlockSpec((1, 1, 128), lambda i: (i, 0, 0)),
        out_shape=jax.ShapeDtypeStruct((nb1, 1, 128), jnp.float32),
        scratch_shapes=[
            pltpu.VMEM((t + _PADT, bb1), jnp.float32),
            pltpu.VMEM((t + _PADT, bb1), jnp.float32),
        ],
        compiler_params=pltpu.CompilerParams(dimension_semantics=("parallel",)),
    )(x)

    lags = pl.pallas_call(
        _topk_kernel,
        in_specs=[pl.BlockSpec((nb1, 1, 128), lambda: (0, 0, 0))],
        out_specs=pl.BlockSpec(memory_space=pltpu.SMEM),
        out_shape=jax.ShapeDtypeStruct((8,), jnp.int32),
    )(parts)

    bb2 = 256
    nb2 = b // bb2
    planes = pl.pallas_call(
        functools.partial(_feat_kernel, bb=bb2, t=t),
        grid_spec=pltpu.PrefetchScalarGridSpec(
            num_scalar_prefetch=1,
            grid=(nb2,),
            in_specs=[pl.BlockSpec((bb2, t), lambda i, lags: (i, 0))],
            out_specs=pl.BlockSpec((_NLAGS + 1, bb2, t), lambda i, lags: (0, i, 0)),
        ),
        out_shape=jax.ShapeDtypeStruct((_NLAGS + 1, b, t), jnp.float32),
        compiler_params=pltpu.CompilerParams(dimension_semantics=("parallel",)),
    )(lags, x)

    return jnp.transpose(planes, (1, 2, 0))
